# baseline (speedup 1.0000x reference)
import functools

import jax
import jax.numpy as jnp
from jax import lax
from jax.experimental import pallas as pl
from jax.experimental.pallas import tpu as pltpu
from jax.experimental.pallas import tpu_sc as plsc

NC = 2
NS = 16
L = 16
NW = NC * NS
CHUNK = 128


def kernel(user_table, item_table, user_idx, item_idx):
    B = user_idx.shape[0]
    D = user_table.shape[1]
    bpw = B // NW
    nchunk = bpw // CHUNK

    ui2 = user_idx.astype(jnp.int32).reshape(NW * nchunk, CHUNK)
    ii2 = item_idx.astype(jnp.int32).reshape(NW * nchunk, CHUNK)

    mesh = plsc.VectorSubcoreMesh(core_axis_name="c", subcore_axis_name="s")

    @functools.partial(
        pl.kernel,
        mesh=mesh,
        compiler_params=pltpu.CompilerParams(
            needs_layout_passes=False, use_tc_tiling_on_sc=False),
        out_type=jax.ShapeDtypeStruct((B,), jnp.float32),
        scratch_types=[
            pltpu.VMEM((nchunk, CHUNK), jnp.int32),
            pltpu.VMEM((nchunk, CHUNK), jnp.int32),
            pltpu.VMEM((bpw, D), jnp.float32),
            pltpu.VMEM((bpw, D), jnp.float32),
            pltpu.VMEM((bpw,), jnp.float32),
            pltpu.SemaphoreType.DMA,
            pltpu.SemaphoreType.DMA,
        ],
    )
    def _k(ut_hbm, it_hbm, ui_hbm, ii_hbm, out_hbm,
           ui_v, ii_v, ur_v, ir_v, o_v, sem_u, sem_i):
        wid = lax.axis_index("s") * NC + lax.axis_index("c")
        pltpu.sync_copy(ui_hbm.at[pl.ds(wid * nchunk, nchunk)], ui_v)
        pltpu.sync_copy(ii_hbm.at[pl.ds(wid * nchunk, nchunk)], ii_v)
        copies = []
        for j in range(nchunk):
            copies.append(pltpu.async_copy(
                ut_hbm.at[ui_v.at[j]], ur_v.at[pl.ds(j * CHUNK, CHUNK)], sem_u))
            copies.append(pltpu.async_copy(
                it_hbm.at[ii_v.at[j]], ir_v.at[pl.ds(j * CHUNK, CHUNK)], sem_i))
        for c in copies:
            c.wait()

        lanes = lax.broadcasted_iota(jnp.int32, (L,), 0)

        def group(g, carry):
            out = jnp.zeros((L,), jnp.float32)
            for k in range(L):
                r = g * L + k
                acc = jnp.zeros((L,), jnp.float32)
                for c in range(D // L):
                    u = ur_v[r, pl.ds(c * L, L)]
                    v = ir_v[r, pl.ds(c * L, L)]
                    acc = acc + u * v
                out = jnp.where(lanes == k, jnp.sum(acc), out)
            o_v[pl.ds(g * L, L)] = out
            return carry

        lax.fori_loop(0, bpw // L, group, 0)
        pltpu.sync_copy(o_v, out_hbm.at[pl.ds(wid * bpw, bpw)])

    return _k(user_table, item_table, ui2, ii2)
